# baseline (speedup 1.0000x reference)
import functools

import jax
import jax.numpy as jnp
from jax import lax
from jax.experimental import pallas as pl
from jax.experimental.pallas import tpu as pltpu
from jax.experimental.pallas import tpu_sc as plsc

B = 16384
D = 32
NC = 2
NS = 16
NW = NC * NS
BPW = B // NW
CHUNK = 128
NCHUNK = BPW // CHUNK
V = 1000000

UB = 2048
RGRID = -(-V // UB)
SUP = UB // 4
VSUP = RGRID * SUP

TC_BLK = 2048
TC_GRID = B // TC_BLK


def _relayout_body(src_ref, dst_ref):
    x = src_ref[...]
    xt = x.T
    dst_ref[...] = jnp.concatenate(
        [xt[k * SUP:(k + 1) * SUP] for k in range(4)], axis=1)


def _relayout(table_t):
    return pl.pallas_call(
        _relayout_body,
        grid=(RGRID,),
        in_specs=[pl.BlockSpec((D, UB), lambda i: (0, i))],
        out_specs=pl.BlockSpec((SUP, 128), lambda i: (i, 0)),
        out_shape=jax.ShapeDtypeStruct((VSUP, 128), jnp.float32),
    )(table_t)


def _gather_body(user_hbm, item_hbm, theta_hbm, a_hbm, b_hbm,
                 th_out, av_out, bv_out,
                 uidx, iidx, su, si, thv, avv, bv, thc, avc, sem):
    wid = lax.axis_index("s") * NC + lax.axis_index("c")
    base = wid * BPW

    pltpu.sync_copy(user_hbm.at[wid], uidx)
    pltpu.sync_copy(item_hbm.at[wid], iidx)

    for j in range(NCHUNK):
        for k in range(CHUNK // 16):
            sl = pl.ds(k * 16, 16)
            u = uidx[j, sl]
            w = lax.shift_right_logical(u, 11)
            su[j, sl] = w * SUP + (u & (SUP - 1))

    for j in range(NCHUNK):
        for k in range(CHUNK // 16):
            sl = pl.ds(k * 16, 16)
            si[j, sl] = lax.shift_right_logical(iidx[j, sl], 2)

    bcopies = [pltpu.async_copy(b_hbm.at[iidx.at[j]],
                                bv.at[pl.ds(j * CHUNK, CHUNK)], sem)
               for j in range(NCHUNK)]

    for h in range(2):
        copies = []
        for jj in range(2):
            j = 2 * h + jj
            sl = pl.ds(jj * CHUNK, CHUNK)
            copies.append(pltpu.async_copy(theta_hbm.at[su.at[j]],
                                           thv.at[sl], sem))
            copies.append(pltpu.async_copy(a_hbm.at[si.at[j]],
                                           avv.at[sl], sem))
        for c in copies:
            c.wait()

        def _extract(g, _):
            uvec = uidx[2 * h + g // 8, pl.ds((g % 8) * 16, 16)]
            ivec = iidx[2 * h + g // 8, pl.ds((g % 8) * 16, 16)]
            for rr in range(16):
                r = g * 16 + rr
                uoff = (uvec[rr] & (UB - 1)) // SUP * D
                ioff = (ivec[rr] & 3) * D
                thc[pl.ds(r * D, 16)] = thv[r, pl.ds(uoff, 16)]
                thc[pl.ds(r * D + 16, 16)] = thv[r, pl.ds(uoff + 16, 16)]
                avc[pl.ds(r * D, 16)] = avv[r, pl.ds(ioff, 16)]
                avc[pl.ds(r * D + 16, 16)] = avv[r, pl.ds(ioff + 16, 16)]
            return 0

        lax.fori_loop(0, 16, _extract, 0)
        osl = pl.ds((base + h * 256) * D, 256 * D)
        pltpu.sync_copy(thc, th_out.at[osl])
        pltpu.sync_copy(avc, av_out.at[osl])

    for c in bcopies:
        c.wait()
    pltpu.sync_copy(bv, bv_out.at[pl.ds(base, BPW)])


def _compute_body(th_ref, av_ref, bv_ref, out_ref):
    t = th_ref[...]
    a = av_ref[...]
    b = bv_ref[...]
    sa = jax.nn.sigmoid(a)
    s = jnp.sum(sa * t, axis=-1)
    out_ref[...] = jax.nn.sigmoid(s.reshape(b.shape) - b)


@jax.jit
def _mirt(user, item, theta_w, a_w, b_flat):
    theta_r = _relayout(theta_w.T)

    mesh = plsc.VectorSubcoreMesh(core_axis_name="c", subcore_axis_name="s",
                                  num_cores=NC, num_subcores=NS)
    gather = functools.partial(
        pl.kernel,
        out_type=(
            jax.ShapeDtypeStruct((B * D,), jnp.float32),
            jax.ShapeDtypeStruct((B * D,), jnp.float32),
            jax.ShapeDtypeStruct((B,), jnp.float32),
        ),
        mesh=mesh,
        compiler_params=pltpu.CompilerParams(use_tc_tiling_on_sc=False),
        scratch_types=[
            pltpu.VMEM((NCHUNK, CHUNK), jnp.int32),
            pltpu.VMEM((NCHUNK, CHUNK), jnp.int32),
            pltpu.VMEM((NCHUNK, CHUNK), jnp.int32),
            pltpu.VMEM((NCHUNK, CHUNK), jnp.int32),
            pltpu.VMEM((256, 128), jnp.float32),
            pltpu.VMEM((256, 128), jnp.float32),
            pltpu.VMEM((BPW,), jnp.float32),
            pltpu.VMEM((256 * D,), jnp.float32),
            pltpu.VMEM((256 * D,), jnp.float32),
            pltpu.SemaphoreType.DMA,
        ],
    )(_gather_body)
    th, av, bv = gather(user.reshape(NW, NCHUNK, CHUNK),
                        item.reshape(NW, NCHUNK, CHUNK),
                        theta_r, a_w.reshape(V // 4, 128), b_flat)

    out = pl.pallas_call(
        _compute_body,
        grid=(TC_GRID,),
        in_specs=[
            pl.BlockSpec((TC_BLK, D), lambda i: (i, 0)),
            pl.BlockSpec((TC_BLK, D), lambda i: (i, 0)),
            pl.BlockSpec((8, TC_BLK // 8), lambda i: (i, 0)),
        ],
        out_specs=pl.BlockSpec((8, TC_BLK // 8), lambda i: (i, 0)),
        out_shape=jax.ShapeDtypeStruct((TC_GRID * 8, TC_BLK // 8), jnp.float32),
    )(th.reshape(B, D), av.reshape(B, D),
      bv.reshape(TC_GRID * 8, TC_BLK // 8))
    return out.reshape(B)


def kernel(user, item, theta_w, a_w, b_w):
    return _mirt(user, item, theta_w, a_w, b_w.reshape(-1))


# speedup vs baseline: 1.0101x; 1.0101x over previous
import functools

import jax
import jax.numpy as jnp
from jax import lax
from jax.experimental import pallas as pl
from jax.experimental.pallas import tpu as pltpu
from jax.experimental.pallas import tpu_sc as plsc

B = 16384
D = 32
NC = 2
NS = 16
NW = NC * NS
BPW = B // NW
CHUNK = 128
NCHUNK = BPW // CHUNK
V = 1000000

UB = 2048
RGRID = -(-V // UB)
SUP = UB // 4
VSUP = RGRID * SUP

TC_BLK = 2048
TC_GRID = B // TC_BLK


def _relayout_body(src_ref, dst_ref):
    x = src_ref[...]
    xt = x.T
    dst_ref[...] = jnp.concatenate(
        [xt[k * SUP:(k + 1) * SUP] for k in range(4)], axis=1)


def _relayout(table_t):
    return pl.pallas_call(
        _relayout_body,
        grid=(RGRID,),
        in_specs=[pl.BlockSpec((D, UB), lambda i: (0, i))],
        out_specs=pl.BlockSpec((SUP, 128), lambda i: (i, 0)),
        out_shape=jax.ShapeDtypeStruct((VSUP, 128), jnp.float32),
    )(table_t)


def _gather_body(user_hbm, item_hbm, theta_hbm, a_hbm, b_hbm,
                 th_out, av_out, bv_out,
                 uidx, iidx, su, thv, avv, bv, thc, sem):
    wid = lax.axis_index("s") * NC + lax.axis_index("c")
    base = wid * BPW

    pltpu.sync_copy(user_hbm.at[wid], uidx)
    pltpu.sync_copy(item_hbm.at[wid], iidx)

    for j in range(NCHUNK):
        for k in range(CHUNK // 16):
            sl = pl.ds(k * 16, 16)
            u = uidx[j, sl]
            w = lax.shift_right_logical(u, 11)
            su[j, sl] = w * SUP + (u & (SUP - 1))

    acopies = []
    for j in range(NCHUNK):
        sl = pl.ds(j * CHUNK, CHUNK)
        acopies.append(pltpu.async_copy(a_hbm.at[iidx.at[j]],
                                        avv.at[sl], sem))
        acopies.append(pltpu.async_copy(b_hbm.at[iidx.at[j]],
                                        bv.at[sl], sem))

    for h in range(2):
        copies = []
        for jj in range(2):
            j = 2 * h + jj
            copies.append(pltpu.async_copy(theta_hbm.at[su.at[j]],
                                           thv.at[pl.ds(jj * CHUNK, CHUNK)],
                                           sem))
        for c in copies:
            c.wait()

        def _extract(g, _):
            uvec = uidx[2 * h + g // 8, pl.ds((g % 8) * 16, 16)]
            for rr in range(16):
                r = g * 16 + rr
                uoff = (uvec[rr] & (UB - 1)) // SUP * D
                thc[pl.ds(r * D, 16)] = thv[r, pl.ds(uoff, 16)]
                thc[pl.ds(r * D + 16, 16)] = thv[r, pl.ds(uoff + 16, 16)]
            return 0

        lax.fori_loop(0, 16, _extract, 0)
        pltpu.sync_copy(thc, th_out.at[pl.ds((base + h * 256) * D, 256 * D)])

    for c in acopies:
        c.wait()
    pltpu.sync_copy(avv, av_out.at[pl.ds(base, BPW)])
    pltpu.sync_copy(bv, bv_out.at[pl.ds(base, BPW)])


def _compute_body(th_ref, av_ref, bv_ref, out_ref):
    t = th_ref[...]
    a = av_ref[...]
    b = bv_ref[...]
    sa = jax.nn.sigmoid(a)
    s = jnp.sum(sa * t, axis=-1)
    out_ref[...] = jax.nn.sigmoid(s.reshape(b.shape) - b)


@jax.jit
def _mirt(user, item, theta_w, a_w, b_flat):
    theta_r = _relayout(theta_w.T)

    mesh = plsc.VectorSubcoreMesh(core_axis_name="c", subcore_axis_name="s",
                                  num_cores=NC, num_subcores=NS)
    gather = functools.partial(
        pl.kernel,
        out_type=(
            jax.ShapeDtypeStruct((B * D,), jnp.float32),
            jax.ShapeDtypeStruct((B, D), jnp.float32),
            jax.ShapeDtypeStruct((B,), jnp.float32),
        ),
        mesh=mesh,
        compiler_params=pltpu.CompilerParams(use_tc_tiling_on_sc=False),
        scratch_types=[
            pltpu.VMEM((NCHUNK, CHUNK), jnp.int32),
            pltpu.VMEM((NCHUNK, CHUNK), jnp.int32),
            pltpu.VMEM((NCHUNK, CHUNK), jnp.int32),
            pltpu.VMEM((256, 128), jnp.float32),
            pltpu.VMEM((BPW, D), jnp.float32),
            pltpu.VMEM((BPW,), jnp.float32),
            pltpu.VMEM((256 * D,), jnp.float32),
            pltpu.SemaphoreType.DMA,
        ],
    )(_gather_body)
    th, av, bv = gather(user.reshape(NW, NCHUNK, CHUNK),
                        item.reshape(NW, NCHUNK, CHUNK),
                        theta_r, a_w, b_flat)

    out = pl.pallas_call(
        _compute_body,
        grid=(TC_GRID,),
        in_specs=[
            pl.BlockSpec((TC_BLK, D), lambda i: (i, 0)),
            pl.BlockSpec((TC_BLK, D), lambda i: (i, 0)),
            pl.BlockSpec((8, TC_BLK // 8), lambda i: (i, 0)),
        ],
        out_specs=pl.BlockSpec((8, TC_BLK // 8), lambda i: (i, 0)),
        out_shape=jax.ShapeDtypeStruct((TC_GRID * 8, TC_BLK // 8), jnp.float32),
    )(th.reshape(B, D), av, bv.reshape(TC_GRID * 8, TC_BLK // 8))
    return out.reshape(B)


def kernel(user, item, theta_w, a_w, b_w):
    return _mirt(user, item, theta_w, a_w, b_w.reshape(-1))
